# baseline (speedup 1.0000x reference)
import functools
import math

import jax
import jax.numpy as jnp
from jax import lax
from jax.experimental import pallas as pl
from jax.experimental.pallas import tpu as pltpu
from jax.experimental.pallas import tpu_sc as plsc

D_MODEL = 768
SCALE = math.sqrt(float(D_MODEL))
LANES = 16
NC, NS = 2, 16
NW = NC * NS
B_TOTAL = 4 * 2048
BPW = B_TOTAL // NW
CHUNK = 32
NCHUNK = BPW // CHUNK
NBUF = 4
PREF = 2
SLICES = D_MODEL // LANES


def _scale_rows(buf):
    @plsc.parallel_loop(0, CHUNK, step=2)
    def _row_body(r):
        for rr in range(2):
            for j in range(SLICES):
                sl = pl.ds(j * LANES, LANES)
                buf[r + rr, sl] = buf[r + rr, sl] * SCALE


BATCH, SEQ = 4, 2048
WPB = NW // BATCH
SPW = SEQ // WPB


def _emb_body(x_hbm, table_hbm, out_hbm, idx_v, bufs, gsems, wsems):
    wid = lax.axis_index("s") * NC + lax.axis_index("c")
    brow = wid // WPB
    s0 = (wid % WPB) * SPW

    pltpu.sync_copy(x_hbm.at[brow, pl.ds(s0, SPW)], idx_v)

    gcopies = [None] * NBUF
    wcopies = [None] * NBUF
    for k in range(NCHUNK + PREF):
        if k < NCHUNK:
            slot = k % NBUF
            if k >= NBUF:
                wcopies[slot].wait()
            gcopies[slot] = pltpu.async_copy(
                table_hbm.at[idx_v.at[pl.ds(k * CHUNK, CHUNK)]],
                bufs[slot],
                gsems[slot],
            )
        if k >= PREF:
            j = k - PREF
            slot = j % NBUF
            gcopies[slot].wait()
            _scale_rows(bufs[slot])
            wcopies[slot] = pltpu.async_copy(
                bufs[slot],
                out_hbm.at[brow, pl.ds(s0 + j * CHUNK, CHUNK)],
                wsems[slot],
            )
    for j in range(NCHUNK - NBUF, NCHUNK):
        wcopies[j % NBUF].wait()


def _emb_entry(x_hbm, table_hbm, out_hbm, *scratch):
    idx_v = scratch[0]
    bufs = scratch[1 : 1 + NBUF]
    gsems = scratch[1 + NBUF : 1 + 2 * NBUF]
    wsems = scratch[1 + 2 * NBUF : 1 + 3 * NBUF]
    _emb_body(x_hbm, table_hbm, out_hbm, idx_v, bufs, gsems, wsems)


_emb = functools.partial(
    pl.kernel,
    out_type=jax.ShapeDtypeStruct((BATCH, SEQ, D_MODEL), jnp.float32),
    mesh=plsc.VectorSubcoreMesh(
        core_axis_name="c", subcore_axis_name="s", num_cores=NC, num_subcores=NS
    ),
    scratch_types=(
        [pltpu.VMEM((BPW,), jnp.int32)]
        + [pltpu.VMEM((CHUNK, D_MODEL), jnp.float32) for _ in range(NBUF)]
        + [pltpu.SemaphoreType.DMA for _ in range(2 * NBUF)]
    ),
)(_emb_entry)


@jax.jit
def kernel(x, table):
    return _emb(x.astype(jnp.int32), table)


# speedup vs baseline: 1.1160x; 1.1160x over previous
import functools
import math

import jax
import jax.numpy as jnp
from jax import lax
from jax.experimental import pallas as pl
from jax.experimental.pallas import tpu as pltpu
from jax.experimental.pallas import tpu_sc as plsc

D_MODEL = 768
SCALE = math.sqrt(float(D_MODEL))
LANES = 16
NC, NS = 2, 16
NW = NC * NS
B_TOTAL = 4 * 2048
BPW = B_TOTAL // NW
CHUNK = 64
NCHUNK = BPW // CHUNK
NBUF = 2
PREF = 1
SLICES = D_MODEL // LANES


def _scale_rows(buf):
    def row_body(r, _):
        for j in range(SLICES):
            sl = pl.ds(j * LANES, LANES)
            buf[r, sl] = buf[r, sl] * SCALE
        return 0

    lax.fori_loop(0, CHUNK, row_body, 0, unroll=False)


BATCH, SEQ = 4, 2048
WPB = NW // BATCH
SPW = SEQ // WPB


def _emb_body(x_hbm, table_hbm, out_hbm, idx_v, bufs, gsems, wsems):
    wid = lax.axis_index("s") * NC + lax.axis_index("c")
    brow = wid // WPB
    s0 = (wid % WPB) * SPW

    pltpu.sync_copy(x_hbm.at[brow, pl.ds(s0, SPW)], idx_v)

    gcopies = [None] * NBUF
    wcopies = [None] * NBUF
    for k in range(NCHUNK + PREF):
        if k < NCHUNK:
            slot = k % NBUF
            if k >= NBUF:
                wcopies[slot].wait()
            gcopies[slot] = pltpu.async_copy(
                table_hbm.at[idx_v.at[pl.ds(k * CHUNK, CHUNK)]],
                bufs[slot],
                gsems[slot],
            )
        if k >= PREF:
            j = k - PREF
            slot = j % NBUF
            gcopies[slot].wait()
            _scale_rows(bufs[slot])
            wcopies[slot] = pltpu.async_copy(
                bufs[slot],
                out_hbm.at[brow, pl.ds(s0 + j * CHUNK, CHUNK)],
                wsems[slot],
            )
    for j in range(NCHUNK - NBUF, NCHUNK):
        wcopies[j % NBUF].wait()


def _emb_entry(x_hbm, table_hbm, out_hbm, *scratch):
    idx_v = scratch[0]
    bufs = scratch[1 : 1 + NBUF]
    gsems = scratch[1 + NBUF : 1 + 2 * NBUF]
    wsems = scratch[1 + 2 * NBUF : 1 + 3 * NBUF]
    _emb_body(x_hbm, table_hbm, out_hbm, idx_v, bufs, gsems, wsems)


_emb = functools.partial(
    pl.kernel,
    out_type=jax.ShapeDtypeStruct((BATCH, SEQ, D_MODEL), jnp.float32),
    mesh=plsc.VectorSubcoreMesh(
        core_axis_name="c", subcore_axis_name="s", num_cores=NC, num_subcores=NS
    ),
    scratch_types=(
        [pltpu.VMEM((BPW,), jnp.int32)]
        + [pltpu.VMEM((CHUNK, D_MODEL), jnp.float32) for _ in range(NBUF)]
        + [pltpu.SemaphoreType.DMA for _ in range(2 * NBUF)]
    ),
)(_emb_entry)


@jax.jit
def kernel(x, table):
    return _emb(x.astype(jnp.int32), table)
